# baseline (speedup 1.0000x reference)
import functools

import jax
import jax.numpy as jnp
from jax import lax
from jax.experimental import pallas as pl
from jax.experimental.pallas import tpu as pltpu
from jax.experimental.pallas import tpu_sc as plsc

SIZE_N = 1000000
D = 64
DP = 128
K = 5
B = 16384
NC = 2
NS = 16
NW = NC * NS
LANES = 16

CW = 1024
NSLAB = 977
CAP_BN = 1024
CAP_BC = 4096
CAP_H = 256
NCTX = B + K * B
DUMP_N = B
DUMP_C = NCTX


def _sc_gather(v_i, v_j, neg_t, nodes_t, ctx_t):
    mesh = plsc.VectorSubcoreMesh(core_axis_name="c", subcore_axis_name="s")

    @functools.partial(
        pl.kernel,
        mesh=mesh,
        compiler_params=pltpu.CompilerParams(needs_layout_passes=False),
        out_type=(
            jax.ShapeDtypeStruct((B + 32, DP), jnp.float32),
            jax.ShapeDtypeStruct((NCTX + 32, DP), jnp.float32),
        ),
        scratch_types=[
            pltpu.VMEM((2048,), jnp.int32),
            pltpu.VMEM((CAP_BN,), jnp.int32),
            pltpu.VMEM((CAP_BN,), jnp.int32),
            pltpu.VMEM((CAP_BC,), jnp.int32),
            pltpu.VMEM((CAP_BC,), jnp.int32),
            pltpu.VMEM((D, CW), jnp.float32),
            pltpu.VMEM((D, LANES), jnp.float32),
            pltpu.VMEM((CAP_H, DP), jnp.float32),
            pltpu.VMEM((CAP_H,), jnp.int32),
            pltpu.VMEM((CAP_H,), jnp.int32),
            pltpu.VMEM((CAP_H // 32, 32), jnp.int32),
            pltpu.SemaphoreType.DMA,
        ],
    )
    def body(vi_hbm, vj_hbm, negt_hbm, nt_hbm, ct_hbm, gvi_hbm, gctx_hbm,
             sbuf, bin_, bpn, bic, bpc, slab, tblock, block, hlc, hpos,
             hpos2, sem):
        w = lax.axis_index("s") * NC + lax.axis_index("c")
        lane = lax.iota(jnp.int32, 16)
        start_slab = w * 30 + jnp.minimum(w, 17)
        n_slab = jnp.where(w < 17, 31, 30)
        lo = start_slab * CW
        hi = (start_slab + n_slab) * CW

        def scan_stream(src_hbm, n_items, pos_base, ib, ipb, off0):
            def chunk_body(c, off):
                pltpu.sync_copy(src_hbm.at[pl.ds(c * 2048, 2048)], sbuf)

                def group_body(g, off2):
                    iv = sbuf[pl.ds(g * 16, 16)]
                    pos = pos_base + c * 2048 + g * 16 + lane
                    m = (iv >= lo) & (iv < hi)
                    mi = m.astype(jnp.int32)
                    dest = off2 + plsc.cumsum(mi) - mi
                    plsc.store_scatter(ib, [dest], iv, mask=m)
                    plsc.store_scatter(ipb, [dest], pos, mask=m)
                    n = plsc.all_reduce_population_count(m)
                    return jnp.minimum(off2 + n, ib.shape[0] - 16)

                return lax.fori_loop(0, 128, group_body, off)

            return lax.fori_loop(0, n_items // 2048, chunk_body, off0)

        zvec = jnp.zeros((16,), jnp.int32)
        nb_n = scan_stream(vi_hbm, B, 0, bin_, bpn, zvec)[0]
        nb_c_v = scan_stream(vj_hbm, B, 0, bic, bpc, zvec)
        nb_c = scan_stream(negt_hbm, K * B, B, bic, bpc, nb_c_v)[0]

        def process_stream(c0, nb, ib, ipb, tbl_hbm, g_hbm, dump):
            @pl.when(c0 < (NSLAB - 1) * CW)
            def _():
                pltpu.sync_copy(
                    tbl_hbm.at[:, pl.ds(pl.multiple_of(c0, 128), CW)], slab)

            @pl.when(c0 >= (NSLAB - 1) * CW)
            def _():
                pltpu.sync_copy(
                    tbl_hbm.at[:, pl.ds(pl.multiple_of(c0, 128), 640)],
                    slab.at[:, pl.ds(0, 640)])

            zero16 = jnp.zeros((16,), jnp.int32)
            dump16 = jnp.full((16,), dump, jnp.int32)
            for q in range(CAP_H // 16):
                hlc[pl.ds(q * 16, 16)] = zero16
                hpos[pl.ds(q * 16, 16)] = dump16

            def compact_body(g, hoff):
                iv = ib[pl.ds(g * 16, 16)]
                p = ipb[pl.ds(g * 16, 16)]
                valid = (g * 16 + lane) < nb
                m = (iv >= c0) & (iv < c0 + CW) & valid
                mi = m.astype(jnp.int32)
                dest = hoff + plsc.cumsum(mi) - mi
                plsc.store_scatter(hlc, [dest], (iv - c0) & (CW - 1), mask=m)
                plsc.store_scatter(hpos, [dest], p, mask=m)
                n = plsc.all_reduce_population_count(m)
                return jnp.minimum(hoff + n, CAP_H - 16)

            nh = lax.fori_loop(0, (nb + 15) // 16, compact_body,
                               jnp.zeros((16,), jnp.int32))[0]

            def hit_group(q, carry):
                lc16 = hlc[pl.ds(q * 16, 16)] & (CW - 1)
                for d in range(D):
                    tblock[d, :] = plsc.load_gather(
                        slab, [jnp.full((16,), d, jnp.int32), lc16])
                for j in range(16):
                    for g4 in range(D // 16):
                        v = plsc.load_gather(
                            tblock, [lane + g4 * 16,
                                     jnp.full((16,), j, jnp.int32)])
                        block[q * 16 + j, pl.ds(g4 * 16, 16)] = v
                return carry

            lax.fori_loop(0, (nh + 15) // 16, hit_group, 0)

            for r in range(CAP_H // 32):
                for cc in range(2):
                    hpos2[r, pl.ds(cc * 16, 16)] = hpos[pl.ds(r * 32 + cc * 16, 16)]

            nch = (nh + 31) // 32

            def fire(ch, carry):
                pltpu.async_copy(block.at[pl.ds(ch * 32, 32)],
                                 g_hbm.at[hpos2.at[ch]], sem)
                return carry

            lax.fori_loop(0, nch, fire, 0)

            def drain(ch, carry):
                pltpu.make_async_copy(block.at[pl.ds(0, 32)],
                                      g_hbm.at[pl.ds(0, 32)], sem).wait()
                return carry

            lax.fori_loop(0, nch, drain, 0)

        def slab_body(m, carry):
            c0 = (start_slab + m) * CW
            process_stream(c0, nb_n, bin_, bpn, nt_hbm, gvi_hbm, DUMP_N)
            process_stream(c0, nb_c, bic, bpc, ct_hbm, gctx_hbm, DUMP_C)
            return carry

        lax.fori_loop(0, n_slab, slab_body, 0)

    return body(v_i, v_j, neg_t, nodes_t, ctx_t)


def _tc_loss(g_vi, g_ctx):
    nblk = 16
    rows = B // nblk

    def body(vi_ref, vj_ref, n0, n1, n2, n3, n4, o_ref):
        i = pl.program_id(0)
        nrefs = (n0, n1, n2, n3, n4)
        xvi = vi_ref[...][:, :D]

        def logsig(x):
            return jnp.minimum(x, 0.0) - jnp.log1p(jnp.exp(-jnp.abs(x)))

        t = jnp.sum(logsig(jnp.sum(xvi * vj_ref[...][:, :D], axis=1)))
        for k in range(K):
            dk = -jnp.sum(xvi * nrefs[k][...][:, :D], axis=1)
            t = t + jnp.sum(logsig(dk))

        @pl.when(i == 0)
        def _():
            o_ref[0, 0] = 0.0

        o_ref[0, 0] += t

        @pl.when(i == nblk - 1)
        def _():
            o_ref[0, 0] = -o_ref[0, 0] / B

    specs = [pl.BlockSpec((rows, DP), lambda i: (i, 0)),
             pl.BlockSpec((rows, DP), lambda i: (i, 0))]
    for k in range(K):
        specs.append(pl.BlockSpec(
            (rows, DP), functools.partial(lambda kk, i: (nblk * (kk + 1) + i, 0), k)))

    return pl.pallas_call(
        body,
        grid=(nblk,),
        in_specs=specs,
        out_shape=jax.ShapeDtypeStruct((1, 1), jnp.float32),
        out_specs=pl.BlockSpec(memory_space=pltpu.SMEM),
    )(g_vi, g_ctx, g_ctx, g_ctx, g_ctx, g_ctx, g_ctx)


def kernel(v_i, v_j, negsamples, device, nodes_embeddings, contextnodes_embeddings):
    vi = v_i.astype(jnp.int32)
    vj = v_j.astype(jnp.int32)
    neg_t = negsamples.astype(jnp.int32).T.reshape(-1)
    g_vi, g_ctx = _sc_gather(vi, vj, neg_t,
                             nodes_embeddings.T, contextnodes_embeddings.T)
    loss = _tc_loss(g_vi, g_ctx)
    return loss[0, 0]


# speedup vs baseline: 1.0011x; 1.0011x over previous
import functools

import jax
import jax.numpy as jnp
from jax import lax
from jax.experimental import pallas as pl
from jax.experimental.pallas import tpu as pltpu
from jax.experimental.pallas import tpu_sc as plsc

SIZE_N = 1000000
D = 64
DP = 128
K = 5
B = 16384
NC = 2
NS = 16
NW = NC * NS
LANES = 16

CW = 1024
NSLAB = 977
CAP_BN = 1024
CAP_BC = 4096
CAP_H = 256
NCTX = B + K * B
DUMP_N = B
DUMP_C = NCTX


def _sc_gather(v_i, v_j, neg_t, nodes_t, ctx_t):
    mesh = plsc.VectorSubcoreMesh(core_axis_name="c", subcore_axis_name="s")

    @functools.partial(
        pl.kernel,
        mesh=mesh,
        compiler_params=pltpu.CompilerParams(needs_layout_passes=False),
        out_type=(
            jax.ShapeDtypeStruct((B + 32, DP), jnp.float32),
            jax.ShapeDtypeStruct((NCTX + 32, DP), jnp.float32),
        ),
        scratch_types=[
            pltpu.VMEM((2048,), jnp.int32),
            pltpu.VMEM((CAP_BN,), jnp.int32),
            pltpu.VMEM((CAP_BN,), jnp.int32),
            pltpu.VMEM((CAP_BC,), jnp.int32),
            pltpu.VMEM((CAP_BC,), jnp.int32),
            pltpu.VMEM((D, CW), jnp.float32),
            pltpu.VMEM((D, LANES), jnp.float32),
            pltpu.VMEM((CAP_H, DP), jnp.float32),
            pltpu.VMEM((CAP_H,), jnp.int32),
            pltpu.VMEM((CAP_H,), jnp.int32),
            pltpu.VMEM((CAP_H // 32, 32), jnp.int32),
            pltpu.SemaphoreType.DMA,
        ],
    )
    def body(vi_hbm, vj_hbm, negt_hbm, nt_hbm, ct_hbm, gvi_hbm, gctx_hbm,
             sbuf, bin_, bpn, bic, bpc, slab, tblock, block, hlc, hpos,
             hpos2, sem):
        w = lax.axis_index("s") * NC + lax.axis_index("c")
        lane = lax.iota(jnp.int32, 16)
        start_slab = w * 30 + jnp.minimum(w, 17)
        n_slab = jnp.where(w < 17, 31, 30)
        lo = start_slab * CW
        hi = (start_slab + n_slab) * CW

        def scan_stream(src_hbm, n_items, pos_base, ib, ipb, off0):
            def chunk_body(c, off):
                pltpu.sync_copy(src_hbm.at[pl.ds(c * 2048, 2048)], sbuf)

                def group_body(g, off2):
                    iv = sbuf[pl.ds(g * 16, 16)]
                    pos = pos_base + c * 2048 + g * 16 + lane
                    m = (iv >= lo) & (iv < hi)
                    mi = m.astype(jnp.int32)
                    dest = off2 + plsc.cumsum(mi) - mi
                    plsc.store_scatter(ib, [dest], iv, mask=m)
                    plsc.store_scatter(ipb, [dest], pos, mask=m)
                    n = plsc.all_reduce_population_count(m)
                    return jnp.minimum(off2 + n, ib.shape[0] - 16)

                return lax.fori_loop(0, 128, group_body, off)

            return lax.fori_loop(0, n_items // 2048, chunk_body, off0)

        zvec = jnp.zeros((16,), jnp.int32)
        nb_n = scan_stream(vi_hbm, B, 0, bin_, bpn, zvec)[0]
        nb_c_v = scan_stream(vj_hbm, B, 0, bic, bpc, zvec)
        nb_c = scan_stream(negt_hbm, K * B, B, bic, bpc, nb_c_v)[0]

        def process_stream(c0, nb, ib, ipb, tbl_hbm, g_hbm, dump):
            @pl.when(c0 < (NSLAB - 1) * CW)
            def _():
                cps = [pltpu.async_copy(
                    tbl_hbm.at[pl.ds(fb * 8, 8),
                               pl.ds(pl.multiple_of(c0, 128), CW)],
                    slab.at[pl.ds(fb * 8, 8)], sem) for fb in range(D // 8)]
                for cp in cps:
                    cp.wait()

            @pl.when(c0 >= (NSLAB - 1) * CW)
            def _():
                pltpu.sync_copy(
                    tbl_hbm.at[:, pl.ds(pl.multiple_of(c0, 128), 640)],
                    slab.at[:, pl.ds(0, 640)])

            zero16 = jnp.zeros((16,), jnp.int32)
            dump16 = jnp.full((16,), dump, jnp.int32)
            for q in range(CAP_H // 16):
                hlc[pl.ds(q * 16, 16)] = zero16
                hpos[pl.ds(q * 16, 16)] = dump16

            def compact_body(g, hoff):
                iv = ib[pl.ds(g * 16, 16)]
                p = ipb[pl.ds(g * 16, 16)]
                valid = (g * 16 + lane) < nb
                m = (iv >= c0) & (iv < c0 + CW) & valid
                mi = m.astype(jnp.int32)
                dest = hoff + plsc.cumsum(mi) - mi
                plsc.store_scatter(hlc, [dest], (iv - c0) & (CW - 1), mask=m)
                plsc.store_scatter(hpos, [dest], p, mask=m)
                n = plsc.all_reduce_population_count(m)
                return jnp.minimum(hoff + n, CAP_H - 16)

            nh = lax.fori_loop(0, (nb + 15) // 16, compact_body,
                               jnp.zeros((16,), jnp.int32))[0]

            def hit_group(q, carry):
                lc16 = hlc[pl.ds(q * 16, 16)] & (CW - 1)
                for d in range(D):
                    tblock[d, :] = plsc.load_gather(
                        slab, [jnp.full((16,), d, jnp.int32), lc16])
                for j in range(16):
                    for g4 in range(D // 16):
                        v = plsc.load_gather(
                            tblock, [lane + g4 * 16,
                                     jnp.full((16,), j, jnp.int32)])
                        block[q * 16 + j, pl.ds(g4 * 16, 16)] = v
                return carry

            lax.fori_loop(0, (nh + 15) // 16, hit_group, 0)

            for r in range(CAP_H // 32):
                for cc in range(2):
                    hpos2[r, pl.ds(cc * 16, 16)] = hpos[pl.ds(r * 32 + cc * 16, 16)]

            nch = (nh + 31) // 32

            def fire(ch, carry):
                pltpu.async_copy(block.at[pl.ds(ch * 32, 32)],
                                 g_hbm.at[hpos2.at[ch]], sem)
                return carry

            lax.fori_loop(0, nch, fire, 0)

            def drain(ch, carry):
                pltpu.make_async_copy(block.at[pl.ds(0, 32)],
                                      g_hbm.at[pl.ds(0, 32)], sem).wait()
                return carry

            lax.fori_loop(0, nch, drain, 0)

        def slab_body(m, carry):
            c0 = (start_slab + m) * CW
            process_stream(c0, nb_n, bin_, bpn, nt_hbm, gvi_hbm, DUMP_N)
            process_stream(c0, nb_c, bic, bpc, ct_hbm, gctx_hbm, DUMP_C)
            return carry

        lax.fori_loop(0, n_slab, slab_body, 0)

    return body(v_i, v_j, neg_t, nodes_t, ctx_t)


def _tc_loss(g_vi, g_ctx):
    nblk = 16
    rows = B // nblk

    def body(vi_ref, vj_ref, n0, n1, n2, n3, n4, o_ref):
        i = pl.program_id(0)
        nrefs = (n0, n1, n2, n3, n4)
        xvi = vi_ref[...][:, :D]

        def logsig(x):
            return jnp.minimum(x, 0.0) - jnp.log1p(jnp.exp(-jnp.abs(x)))

        t = jnp.sum(logsig(jnp.sum(xvi * vj_ref[...][:, :D], axis=1)))
        for k in range(K):
            dk = -jnp.sum(xvi * nrefs[k][...][:, :D], axis=1)
            t = t + jnp.sum(logsig(dk))

        @pl.when(i == 0)
        def _():
            o_ref[0, 0] = 0.0

        o_ref[0, 0] += t

        @pl.when(i == nblk - 1)
        def _():
            o_ref[0, 0] = -o_ref[0, 0] / B

    specs = [pl.BlockSpec((rows, DP), lambda i: (i, 0)),
             pl.BlockSpec((rows, DP), lambda i: (i, 0))]
    for k in range(K):
        specs.append(pl.BlockSpec(
            (rows, DP), functools.partial(lambda kk, i: (nblk * (kk + 1) + i, 0), k)))

    return pl.pallas_call(
        body,
        grid=(nblk,),
        in_specs=specs,
        out_shape=jax.ShapeDtypeStruct((1, 1), jnp.float32),
        out_specs=pl.BlockSpec(memory_space=pltpu.SMEM),
    )(g_vi, g_ctx, g_ctx, g_ctx, g_ctx, g_ctx, g_ctx)


def kernel(v_i, v_j, negsamples, device, nodes_embeddings, contextnodes_embeddings):
    vi = v_i.astype(jnp.int32)
    vj = v_j.astype(jnp.int32)
    neg_t = negsamples.astype(jnp.int32).T.reshape(-1)
    g_vi, g_ctx = _sc_gather(vi, vj, neg_t,
                             nodes_embeddings.T, contextnodes_embeddings.T)
    loss = _tc_loss(g_vi, g_ctx)
    return loss[0, 0]


# speedup vs baseline: 1.1779x; 1.1766x over previous
import functools

import jax
import jax.numpy as jnp
from jax import lax
from jax.experimental import pallas as pl
from jax.experimental.pallas import tpu as pltpu
from jax.experimental.pallas import tpu_sc as plsc

D = 64
DP = 128
K = 5
NC = 2
NS = 16
NW = NC * NS
LANES = 16


def _sc_vi_gather(v_i, nodes2):
    B = v_i.shape[0]
    PB = B // NW
    C = min(128, PB)
    NCHUNK = PB // C

    mesh = plsc.VectorSubcoreMesh(core_axis_name="c", subcore_axis_name="s")

    @functools.partial(
        pl.kernel,
        mesh=mesh,
        compiler_params=pltpu.CompilerParams(needs_layout_passes=False),
        out_type=jax.ShapeDtypeStruct((B, DP), jnp.float32),
        scratch_types=[
            pltpu.VMEM((C,), jnp.int32),
            pltpu.VMEM((C,), jnp.int32),
            pltpu.VMEM((C, DP), jnp.float32),
            pltpu.SemaphoreType.DMA,
        ],
    )
    def body(vi_hbm, nodes_hbm, out_hbm, vi_idx, vi_q, rows, sem):
        wid = lax.axis_index("s") * NC + lax.axis_index("c")

        def chunk_body(ci, carry):
            base = wid * PB + ci * C
            pltpu.sync_copy(vi_hbm.at[pl.ds(base, C)], vi_idx)

            def gb(g, carry2):
                vi_q[pl.ds(g * 16, 16)] = vi_idx[pl.ds(g * 16, 16)] >> 1
                return carry2

            lax.fori_loop(0, C // 16, gb, 0)
            pltpu.async_copy(nodes_hbm.at[vi_q], rows, sem).wait()
            pltpu.sync_copy(rows, out_hbm.at[pl.ds(base, C)])
            return carry

        lax.fori_loop(0, NCHUNK, chunk_body, 0)

    return body(v_i, nodes2)


def _sc_dots(v_i, v_j, neg_t, gvi, ctx2):
    B = v_i.shape[0]
    PB = B // NW
    C = min(128, PB)
    NCHUNK = PB // C

    mesh = plsc.VectorSubcoreMesh(core_axis_name="c", subcore_axis_name="s")

    @functools.partial(
        pl.kernel,
        mesh=mesh,
        compiler_params=pltpu.CompilerParams(needs_layout_passes=False),
        out_type=jax.ShapeDtypeStruct((1 + K, B), jnp.float32),
        scratch_types=[
            pltpu.VMEM((C,), jnp.int32),
            pltpu.VMEM((C,), jnp.int32),
            pltpu.VMEM((K * C,), jnp.int32),
            pltpu.VMEM((C,), jnp.int32),
            pltpu.VMEM((C,), jnp.int32),
            pltpu.VMEM((K * C,), jnp.int32),
            pltpu.VMEM((C, DP), jnp.float32),
            pltpu.VMEM((C, DP), jnp.float32),
            pltpu.VMEM((K * C, DP), jnp.float32),
            pltpu.VMEM((1 + K, C), jnp.float32),
            pltpu.SemaphoreType.DMA,
        ],
    )
    def body(vi_hbm, vj_hbm, negt_hbm, gvi_hbm, ctx_hbm, out_hbm,
             vi_idx, vj_idx, neg_idx, vi_q, vj_q, neg_q,
             vi_rows, vj_rows, neg_rows, dots, sem):
        wid = lax.axis_index("s") * NC + lax.axis_index("c")
        lane = lax.iota(jnp.int32, 16)

        def split_q(idx_ref, q_ref, n):
            def gb(g, carry):
                q_ref[pl.ds(g * 16, 16)] = idx_ref[pl.ds(g * 16, 16)] >> 1
                return carry
            lax.fori_loop(0, n // 16, gb, 0)

        def chunk_body(ci, carry):
            base = wid * PB + ci * C
            pltpu.sync_copy(vi_hbm.at[pl.ds(base, C)], vi_idx)
            pltpu.sync_copy(vj_hbm.at[pl.ds(base, C)], vj_idx)
            for k in range(K):
                pltpu.sync_copy(negt_hbm.at[pl.ds(k * B + base, C)],
                                neg_idx.at[pl.ds(k * C, C)])
            split_q(vj_idx, vj_q, C)
            split_q(neg_idx, neg_q, K * C)
            copies = [
                pltpu.make_async_copy(gvi_hbm.at[pl.ds(base, C)], vi_rows, sem),
                pltpu.async_copy(ctx_hbm.at[vj_q], vj_rows, sem),
            ]
            copies[0].start()
            for k in range(K):
                copies.append(
                    pltpu.async_copy(ctx_hbm.at[neg_q.at[pl.ds(k * C, C)]],
                                     neg_rows.at[pl.ds(k * C, C)], sem))
            for c in copies:
                c.wait()

            def group_body(g, carry2):
                elem = g * 16 + lane
                off_i = (vi_idx[pl.ds(g * 16, 16)] & 1) * D
                off_j = (vj_idx[pl.ds(g * 16, 16)] & 1) * D
                off_n = [(neg_idx[pl.ds(k * C + g * 16, 16)] & 1) * D
                         for k in range(K)]
                nelem = [k * C + g * 16 + lane for k in range(K)]
                pos = jnp.zeros((16,), jnp.float32)
                neg = [jnp.zeros((16,), jnp.float32) for _ in range(K)]
                for w in range(D):
                    vv = plsc.load_gather(vi_rows, [elem, off_i + w])
                    jv = plsc.load_gather(vj_rows, [elem, off_j + w])
                    pos = pos + vv * jv
                    for k in range(K):
                        nv = plsc.load_gather(neg_rows, [nelem[k], off_n[k] + w])
                        neg[k] = neg[k] - vv * nv
                dots[0, pl.ds(g * 16, 16)] = pos
                for k in range(K):
                    dots[1 + k, pl.ds(g * 16, 16)] = neg[k]
                return carry2

            lax.fori_loop(0, C // 16, group_body, 0)
            pltpu.sync_copy(
                dots, out_hbm.at[:, pl.ds(pl.multiple_of(base, 128), C)])
            return carry

        lax.fori_loop(0, NCHUNK, chunk_body, 0)

    return body(v_i, v_j, neg_t, gvi, ctx2)


def _tc_loss(dots2d, batch):
    def body(x_ref, o_ref):
        x = x_ref[...]
        ls = jnp.minimum(x, 0.0) - jnp.log1p(jnp.exp(-jnp.abs(x)))
        o_ref[0, 0] = -jnp.sum(ls) / batch

    return pl.pallas_call(
        body,
        out_shape=jax.ShapeDtypeStruct((1, 1), jnp.float32),
        out_specs=pl.BlockSpec(memory_space=pltpu.SMEM),
    )(dots2d)


def kernel(v_i, v_j, negsamples, device, nodes_embeddings, contextnodes_embeddings):
    B = v_i.shape[0]
    vi = v_i.astype(jnp.int32)
    vj = v_j.astype(jnp.int32)
    neg_t = negsamples.astype(jnp.int32).T.reshape(-1)
    n2 = nodes_embeddings.reshape(-1, DP)
    c2 = contextnodes_embeddings.reshape(-1, DP)
    gvi = _sc_vi_gather(vi, n2)
    dots = _sc_dots(vi, vj, neg_t, gvi, c2)
    loss = _tc_loss(dots.reshape((1 + K) * B // 128, 128), B)
    return loss[0, 0]


# speedup vs baseline: 1.3571x; 1.1521x over previous
import functools

import jax
import jax.numpy as jnp
from jax import lax
from jax.experimental import pallas as pl
from jax.experimental.pallas import tpu as pltpu
from jax.experimental.pallas import tpu_sc as plsc

D = 64
DP = 128
K = 5
NC = 2
NS = 16
NW = NC * NS
LANES = 16
PAD = 1.0e9


def _sc_dots(v_i, v_j, neg_t, nodes_p, ctx_p):
    B = v_i.shape[0]
    PB = B // NW
    C = min(128, PB)
    NCHUNK = PB // C
    CR = C // 8

    mesh = plsc.VectorSubcoreMesh(core_axis_name="c", subcore_axis_name="s")

    @functools.partial(
        pl.kernel,
        mesh=mesh,
        out_type=jax.ShapeDtypeStruct((B // 8, 128), jnp.float32),
        scratch_types=[
            pltpu.VMEM((C,), jnp.int32),
            pltpu.VMEM((C,), jnp.int32),
            pltpu.VMEM((K * C,), jnp.int32),
            pltpu.VMEM((C, DP), jnp.float32),
            pltpu.VMEM((C, DP), jnp.float32),
            pltpu.VMEM((K * C, DP), jnp.float32),
            pltpu.VMEM((CR, 128), jnp.float32),
            pltpu.SemaphoreType.DMA,
        ],
    )
    def body(vi_hbm, vj_hbm, negt_hbm, nodes_hbm, ctx_hbm, out_hbm,
             vi_idx, vj_idx, neg_idx, vi_rows, vj_rows, neg_rows, dots, sem):
        wid = lax.axis_index("s") * NC + lax.axis_index("c")
        lane = lax.iota(jnp.int32, 16)
        pad_vec = jnp.where(lane < 1 + K, jnp.float32(0), jnp.float32(PAD))

        def chunk_body(ci, carry):
            base = wid * PB + ci * C
            pltpu.sync_copy(vi_hbm.at[pl.ds(base, C)], vi_idx)
            pltpu.sync_copy(vj_hbm.at[pl.ds(base, C)], vj_idx)
            for k in range(K):
                pltpu.sync_copy(negt_hbm.at[pl.ds(k * B + base, C)],
                                neg_idx.at[pl.ds(k * C, C)])
            copies = [
                pltpu.async_copy(nodes_hbm.at[vi_idx], vi_rows, sem),
                pltpu.async_copy(ctx_hbm.at[vj_idx], vj_rows, sem),
            ]
            for k in range(K):
                copies.append(
                    pltpu.async_copy(ctx_hbm.at[neg_idx.at[pl.ds(k * C, C)]],
                                     neg_rows.at[pl.ds(k * C, C)], sem))
            for c in copies:
                c.wait()

            def lane_sum(x):
                for sh in (8, 4, 2, 1):
                    x = x + x.at[lane ^ sh].get(mode="promise_in_bounds")
                return x

            def elem_body(i, carry2):
                vi_g = [vi_rows[i, pl.ds(g * LANES, LANES)] for g in range(D // LANES)]
                acc = vi_g[0] * vj_rows[i, pl.ds(0, LANES)]
                for g in range(1, D // LANES):
                    acc += vi_g[g] * vj_rows[i, pl.ds(g * LANES, LANES)]
                dvec = jnp.where(lane == 0, lane_sum(acc), pad_vec)
                for k in range(K):
                    nacc = vi_g[0] * neg_rows[k * C + i, pl.ds(0, LANES)]
                    for g in range(1, D // LANES):
                        nacc += vi_g[g] * neg_rows[k * C + i, pl.ds(g * LANES, LANES)]
                    dvec = jnp.where(lane == k + 1, -lane_sum(nacc), dvec)
                dots[i // 8, pl.ds((i % 8) * LANES, LANES)] = dvec
                return carry2

            lax.fori_loop(0, C, elem_body, 0)
            pltpu.sync_copy(
                dots, out_hbm.at[pl.ds(pl.multiple_of(base // 8, 8), CR)])
            return carry

        lax.fori_loop(0, NCHUNK, chunk_body, 0)

    return body(v_i, v_j, neg_t, nodes_p, ctx_p)


def _tc_loss(dots2d, batch):
    def body(x_ref, o_ref):
        x = x_ref[...]
        ls = jnp.minimum(x, 0.0) - jnp.log1p(jnp.exp(-jnp.abs(x)))
        o_ref[0, 0] = -jnp.sum(ls) / batch

    return pl.pallas_call(
        body,
        out_shape=jax.ShapeDtypeStruct((1, 1), jnp.float32),
        out_specs=pl.BlockSpec(memory_space=pltpu.SMEM),
    )(dots2d)


def kernel(v_i, v_j, negsamples, device, nodes_embeddings, contextnodes_embeddings):
    B = v_i.shape[0]
    vi = v_i.astype(jnp.int32)
    vj = v_j.astype(jnp.int32)
    neg_t = negsamples.astype(jnp.int32).T.reshape(-1)
    nodes_p = jnp.pad(nodes_embeddings, ((0, 0), (0, DP - D)))
    ctx_p = jnp.pad(contextnodes_embeddings, ((0, 0), (0, DP - D)))
    dots = _sc_dots(vi, vj, neg_t, nodes_p, ctx_p)
    loss = _tc_loss(dots, B)
    return loss[0, 0]
